# baseline (speedup 1.0000x reference)
import functools

import jax
import jax.numpy as jnp
import numpy as np
from jax import lax
from jax.experimental import pallas as pl
from jax.experimental.pallas import tpu as pltpu
from jax.experimental.pallas import tpu_sc as plsc

_BF = 96.0
_DIST = np.array([0.5, 1, 1.5, 2, 5, 10, 20, 30, 50, 100], dtype=np.float64)
_INV = _BF / _DIST
_NB = 10
_TLO = float(_INV[-1])
_THI = float(_INV[0])
_LN2 = float(np.log(2.0))

_af = np.zeros(_NB); _bf = np.zeros(_NB)
_ar = np.zeros(_NB); _br = np.zeros(_NB)
for _j in range(_NB):
    if _j <= _NB - 2:
        _s = 1.0 / (_INV[_j] - _INV[_j + 1])
        _af[_j] = _s
        _bf[_j] = -_INV[_j + 1] * _s
    else:
        _af[_j] = 0.0
        _bf[_j] = 1.0
    if _j >= 1:
        _s = 1.0 / (_INV[_j - 1] - _INV[_j])
        _ar[_j] = -_s
        _br[_j] = _INV[_j - 1] * _s
    else:
        _ar[_j] = 0.0
        _br[_j] = 1.0

_CAF = np.broadcast_to(_af[:, None], (_NB, 128)).astype(np.float32).copy()
_CBF = np.broadcast_to(_bf[:, None], (_NB, 128)).astype(np.float32).copy()
_CAR = np.broadcast_to(_ar[:, None], (_NB, 128)).astype(np.float32).copy()
_CBR = np.broadcast_to(_br[:, None], (_NB, 128)).astype(np.float32).copy()

_PA = np.zeros(_NB - 1); _PB = np.zeros(_NB - 1)
for _k in range(_NB - 1):
    _s = 1.0 / (_INV[_k] - _INV[_k + 1])
    _PA[_k] = -_s
    _PB[_k] = _INV[_k] * _s

_mm = np.linspace(1.0, 2.0, 4001)
_L2P = np.polyfit(_mm, np.log2(_mm), 4).astype(np.float32)

_BR = 128
_TCBLK = 128 * _BR


def _tc_body(p_ref, t_ref, af_ref, bf_ref, ar_ref, br_ref, o_ref, acc_ref):
    i = pl.program_id(0)
    ni = pl.num_programs(0)

    @pl.when(i == 0)
    def _():
        acc_ref[...] = jnp.zeros_like(acc_ref)

    af = af_ref[...]
    bf = bf_ref[...]
    ar = ar_ref[...]
    br = br_ref[...]
    tc = jnp.clip(t_ref[...], _TLO, _THI)
    accs = [jnp.zeros((_NB, 128), jnp.float32) for _ in range(4)]
    for k in range(_BR):
        tk = tc[k:k + 1, :]
        w = jnp.maximum(jnp.minimum(af * tk + bf, ar * tk + br), 0.0)
        p = p_ref[:, 128 * k:128 * (k + 1)]
        g1 = jnp.log2(p)
        g0 = jnp.log2(1.0 - p)
        accs[k % 4] = accs[k % 4] + (g0 + w * (g1 - g0))
    acc_ref[...] += (accs[0] + accs[1]) + (accs[2] + accs[3])

    @pl.when(i == ni - 1)
    def _():
        o_ref[0, 0] = jnp.sum(acc_ref[...])


def _tc_sum(pt, t2, tc_lanes):
    return pl.pallas_call(
        _tc_body,
        grid=(tc_lanes // _TCBLK,),
        in_specs=[
            pl.BlockSpec((_NB, _TCBLK), lambda i: (0, i)),
            pl.BlockSpec((_BR, 128), lambda i: (i, 0)),
            pl.BlockSpec((_NB, 128), lambda i: (0, 0)),
            pl.BlockSpec((_NB, 128), lambda i: (0, 0)),
            pl.BlockSpec((_NB, 128), lambda i: (0, 0)),
            pl.BlockSpec((_NB, 128), lambda i: (0, 0)),
        ],
        out_specs=pl.BlockSpec(
            (1, 1), lambda i: (0, 0), memory_space=pltpu.SMEM),
        out_shape=jax.ShapeDtypeStruct((1, 1), jnp.float32),
        scratch_shapes=[pltpu.VMEM((_NB, 128), jnp.float32)],
    )(pt, t2, jnp.asarray(_CAF), jnp.asarray(_CBF),
      jnp.asarray(_CAR), jnp.asarray(_CBR))


_NW = 32


def _sclog2(x):
    bits = lax.bitcast_convert_type(x, jnp.int32)
    e = lax.shift_right_arithmetic(bits, 23) - 127
    m = lax.bitcast_convert_type(
        (bits & 0x007FFFFF) | 0x3F800000, jnp.float32)
    r = jnp.full((16,), float(_L2P[0]), jnp.float32)
    for c in _L2P[1:]:
        r = r * m + jnp.float32(c)
    return e.astype(jnp.float32) + r


def _make_sc_sum(sc_lanes, sc_base):
    mw = sc_lanes // _NW
    scmesh = plsc.VectorSubcoreMesh(core_axis_name="c", subcore_axis_name="s")

    @functools.partial(
        pl.kernel,
        out_type=jax.ShapeDtypeStruct((_NW, 16), jnp.float32),
        mesh=scmesh,
        scratch_types=[
            pltpu.VMEM((8, mw), jnp.float32),
            pltpu.VMEM((2, mw), jnp.float32),
            pltpu.VMEM((mw,), jnp.float32),
            pltpu.VMEM((16,), jnp.float32),
        ],
    )
    def sc_sum(pred_hbm, t_hbm, out_hbm, p8, p2, tb, ob):
        wid = lax.axis_index("s") * 2 + lax.axis_index("c")
        base = sc_base + wid * mw
        pltpu.sync_copy(t_hbm.at[pl.ds(base, mw)], tb)
        pltpu.sync_copy(pred_hbm.at[0:8, pl.ds(base, mw)], p8)
        pltpu.sync_copy(pred_hbm.at[8:10, pl.ds(base, mw)], p2)

        def group(i, acc):
            tv = jnp.clip(tb[pl.ds(i * 16, 16)], _TLO, _THI)
            pos = jnp.zeros((16,), jnp.float32)
            for k in range(_NB - 1):
                seg = jnp.float32(_PA[k]) * tv + jnp.float32(_PB[k])
                pos = pos + jnp.minimum(jnp.maximum(seg, 0.0), 1.0)
            acc2 = acc
            for j in range(_NB):
                src = p8 if j < 8 else p2
                pv = src.at[j if j < 8 else j - 8][pl.ds(i * 16, 16)]
                g1 = _sclog2(pv)
                g0 = _sclog2(1.0 - pv)
                w = jnp.maximum(1.0 - jnp.abs(pos - jnp.float32(j)), 0.0)
                acc2 = acc2 + (g0 + w * (g1 - g0))
            return acc2

        acc = lax.fori_loop(0, mw // 16, group, jnp.zeros((16,), jnp.float32))
        ob[...] = acc
        pltpu.sync_copy(ob, out_hbm.at[wid])

    return sc_sum


def kernel(pred_cost, inv_dist_true):
    n, ncol = pred_cost.shape
    pt = pred_cost.T
    t2 = inv_dist_true.reshape(n // 128, 128)
    tf = inv_dist_true.reshape(-1)
    nblk = n // _TCBLK
    sc_blk = (nblk * 11 + 32) // 64
    sc_lanes = sc_blk * _TCBLK
    tc_lanes = n - sc_lanes
    tc_out = _tc_sum(pt, t2, tc_lanes)
    sc_part = _make_sc_sum(sc_lanes, tc_lanes)(pt, tf)
    total = tc_out[0, 0] + jnp.sum(sc_part)
    return total * jnp.float32(-_LN2) / jnp.float32(n * ncol)


# speedup vs baseline: 1.8029x; 1.8029x over previous
import jax
import jax.numpy as jnp
import numpy as np
from jax.experimental import pallas as pl
from jax.experimental.pallas import tpu as pltpu

_BF = 96.0
_DIST = np.array([0.5, 1, 1.5, 2, 5, 10, 20, 30, 50, 100], dtype=np.float64)
_INV = _BF / _DIST
_NB = 10
_TLO = float(_INV[-1])
_THI = float(_INV[0])

_af = np.zeros(_NB); _bf = np.zeros(_NB)
_ar = np.zeros(_NB); _br = np.zeros(_NB)
for _j in range(_NB):
    if _j <= _NB - 2:
        _s = 1.0 / (_INV[_j] - _INV[_j + 1])
        _af[_j] = _s
        _bf[_j] = -_INV[_j + 1] * _s
    else:
        _af[_j] = 0.0
        _bf[_j] = 1.0
    if _j >= 1:
        _s = 1.0 / (_INV[_j - 1] - _INV[_j])
        _ar[_j] = -_s
        _br[_j] = _INV[_j - 1] * _s
    else:
        _ar[_j] = 0.0
        _br[_j] = 1.0

_CAF = np.broadcast_to(_af[:, None], (_NB, 256)).astype(np.float32).copy()
_CBF = np.broadcast_to(_bf[:, None], (_NB, 256)).astype(np.float32).copy()
_CAR = np.broadcast_to(_ar[:, None], (_NB, 256)).astype(np.float32).copy()
_CBR = np.broadcast_to(_br[:, None], (_NB, 256)).astype(np.float32).copy()

_BR = 128
_LN2 = float(np.log(2.0))


def _body(p_ref, t_ref, af_ref, bf_ref, ar_ref, br_ref, o_ref, acc_ref):
    i = pl.program_id(0)
    ni = pl.num_programs(0)

    @pl.when(i == 0)
    def _():
        acc_ref[...] = jnp.zeros_like(acc_ref)

    af = af_ref[...].astype(jnp.bfloat16)
    bf = bf_ref[...].astype(jnp.bfloat16)
    ar = ar_ref[...].astype(jnp.bfloat16)
    br = br_ref[...].astype(jnp.bfloat16)
    tc = jnp.clip(t_ref[...], _TLO, _THI)
    accs = [jnp.zeros((_NB, 256), jnp.float32) for _ in range(4)]
    for k in range(_BR // 2):
        tk = jnp.concatenate(
            [tc[2 * k:2 * k + 1, :], tc[2 * k + 1:2 * k + 2, :]],
            axis=1).astype(jnp.bfloat16)
        w = jnp.maximum(jnp.minimum(af * tk + bf, ar * tk + br),
                        jnp.bfloat16(0.0))
        pf = p_ref[:, 256 * k:256 * (k + 1)]
        g1 = jnp.log2(pf).astype(jnp.bfloat16)
        g0 = jnp.log2(1.0 - pf).astype(jnp.bfloat16)
        c = g0 + w * (g1 - g0)
        accs[k % 4] = accs[k % 4] + c.astype(jnp.float32)
    acc_ref[...] += (accs[0] + accs[1]) + (accs[2] + accs[3])

    @pl.when(i == ni - 1)
    def _():
        o_ref[0, 0] = jnp.sum(acc_ref[...]) * _LN2


def kernel(pred_cost, inv_dist_true):
    n, ncol = pred_cost.shape
    pt = pred_cost.T
    t2 = inv_dist_true.reshape(n // 128, 128)
    blk = 128 * _BR
    out = pl.pallas_call(
        _body,
        grid=(n // blk,),
        in_specs=[
            pl.BlockSpec((ncol, blk), lambda i: (0, i)),
            pl.BlockSpec((_BR, 128), lambda i: (i, 0)),
            pl.BlockSpec((ncol, 256), lambda i: (0, 0)),
            pl.BlockSpec((ncol, 256), lambda i: (0, 0)),
            pl.BlockSpec((ncol, 256), lambda i: (0, 0)),
            pl.BlockSpec((ncol, 256), lambda i: (0, 0)),
        ],
        out_specs=pl.BlockSpec(
            (1, 1), lambda i: (0, 0), memory_space=pltpu.SMEM),
        out_shape=jax.ShapeDtypeStruct((1, 1), jnp.float32),
        scratch_shapes=[pltpu.VMEM((_NB, 256), jnp.float32)],
    )(pt, t2, jnp.asarray(_CAF), jnp.asarray(_CBF),
      jnp.asarray(_CAR), jnp.asarray(_CBR))
    return -out[0, 0] / jnp.float32(n * ncol)


# speedup vs baseline: 1.8051x; 1.0012x over previous
import jax
import jax.numpy as jnp
import numpy as np
from jax.experimental import pallas as pl
from jax.experimental.pallas import tpu as pltpu

_BF = 96.0
_DIST = np.array([0.5, 1, 1.5, 2, 5, 10, 20, 30, 50, 100], dtype=np.float64)
_INV = _BF / _DIST
_NB = 10
_TLO = float(_INV[-1])
_THI = float(_INV[0])

_af = np.zeros(_NB); _bf = np.zeros(_NB)
_ar = np.zeros(_NB); _br = np.zeros(_NB)
for _j in range(_NB):
    if _j <= _NB - 2:
        _s = 1.0 / (_INV[_j] - _INV[_j + 1])
        _af[_j] = _s
        _bf[_j] = -_INV[_j + 1] * _s
    else:
        _af[_j] = 0.0
        _bf[_j] = 1.0
    if _j >= 1:
        _s = 1.0 / (_INV[_j - 1] - _INV[_j])
        _ar[_j] = -_s
        _br[_j] = _INV[_j - 1] * _s
    else:
        _ar[_j] = 0.0
        _br[_j] = 1.0

_CAF = np.broadcast_to(_af[:, None], (_NB, 256)).astype(np.float32).copy()
_CBF = np.broadcast_to(_bf[:, None], (_NB, 256)).astype(np.float32).copy()
_CAR = np.broadcast_to(_ar[:, None], (_NB, 256)).astype(np.float32).copy()
_CBR = np.broadcast_to(_br[:, None], (_NB, 256)).astype(np.float32).copy()

_BR = 512
_LN2 = float(np.log(2.0))


def _body(p_ref, t_ref, af_ref, bf_ref, ar_ref, br_ref, o_ref, acc_ref):
    i = pl.program_id(0)
    ni = pl.num_programs(0)

    @pl.when(i == 0)
    def _():
        acc_ref[...] = jnp.zeros_like(acc_ref)

    af = af_ref[...].astype(jnp.bfloat16)
    bf = bf_ref[...].astype(jnp.bfloat16)
    ar = ar_ref[...].astype(jnp.bfloat16)
    br = br_ref[...].astype(jnp.bfloat16)
    tc = jnp.clip(t_ref[...], _TLO, _THI)
    accs = [jnp.zeros((_NB, 256), jnp.float32) for _ in range(4)]
    for k in range(_BR // 2):
        tk = jnp.concatenate(
            [tc[2 * k:2 * k + 1, :], tc[2 * k + 1:2 * k + 2, :]],
            axis=1).astype(jnp.bfloat16)
        w = jnp.maximum(jnp.minimum(af * tk + bf, ar * tk + br),
                        jnp.bfloat16(0.0))
        pf = p_ref[:, 256 * k:256 * (k + 1)]
        g1 = jnp.log2(pf).astype(jnp.bfloat16)
        g0 = jnp.log2(1.0 - pf).astype(jnp.bfloat16)
        c = g0 + w * (g1 - g0)
        accs[k % 4] = accs[k % 4] + c.astype(jnp.float32)
    acc_ref[...] += (accs[0] + accs[1]) + (accs[2] + accs[3])

    @pl.when(i == ni - 1)
    def _():
        o_ref[0, 0] = jnp.sum(acc_ref[...]) * _LN2


def kernel(pred_cost, inv_dist_true):
    n, ncol = pred_cost.shape
    pt = pred_cost.T
    t2 = inv_dist_true.reshape(n // 128, 128)
    blk = 128 * _BR
    out = pl.pallas_call(
        _body,
        grid=(n // blk,),
        in_specs=[
            pl.BlockSpec((ncol, blk), lambda i: (0, i)),
            pl.BlockSpec((_BR, 128), lambda i: (i, 0)),
            pl.BlockSpec((ncol, 256), lambda i: (0, 0)),
            pl.BlockSpec((ncol, 256), lambda i: (0, 0)),
            pl.BlockSpec((ncol, 256), lambda i: (0, 0)),
            pl.BlockSpec((ncol, 256), lambda i: (0, 0)),
        ],
        out_specs=pl.BlockSpec(
            (1, 1), lambda i: (0, 0), memory_space=pltpu.SMEM),
        out_shape=jax.ShapeDtypeStruct((1, 1), jnp.float32),
        scratch_shapes=[pltpu.VMEM((_NB, 256), jnp.float32)],
    )(pt, t2, jnp.asarray(_CAF), jnp.asarray(_CBF),
      jnp.asarray(_CAR), jnp.asarray(_CBR))
    return -out[0, 0] / jnp.float32(n * ncol)
